# baseline (speedup 1.0000x reference)
import functools

import jax
import jax.numpy as jnp
from jax import lax
from jax.experimental import pallas as pl
from jax.experimental.pallas import tpu as pltpu
from jax.experimental.pallas import tpu_sc as plsc

_GROUPS = ((1, 2), (3, 4), (5, 6), (7, 8), (0, 9),
           (10, 11, 12), (13, 14), (15, 16), (17, 18), (19, 20))
_J = 21
_G = 10
_C = 128
_LANES = 16
_NW = 32
_R = 8
_NBUF = 4


def _make_kernel(B, T):
    b_per_w = B // _NW
    t_steps = T // _R
    steps = b_per_w * t_steps
    assert steps % _NBUF == 0
    mesh = plsc.VectorSubcoreMesh(core_axis_name="c", subcore_axis_name="s")

    @functools.partial(
        pl.kernel,
        out_type=jax.ShapeDtypeStruct((B, _G, T, _C), jnp.float32),
        mesh=mesh,
        scratch_types=[
            pltpu.VMEM((_NBUF, _J, _R, _C), jnp.float32),
            pltpu.VMEM((_NBUF, _G, _R, _C), jnp.float32),
        ] + [pltpu.SemaphoreType.DMA] * (2 * _NBUF),
    )
    def k(x_hbm, out_hbm, in_v, out_v, *sems):
        sin = sems[:_NBUF]
        sout = sems[_NBUF:]
        wid = lax.axis_index("s") * 2 + lax.axis_index("c")
        base_b = wid * b_per_w

        def in_copy(step, b):
            bb = base_b + step // t_steps
            t0 = (step % t_steps) * _R
            return pltpu.make_async_copy(
                x_hbm.at[bb, :, pl.ds(t0, _R)], in_v.at[b], sin[b])

        def out_copy(step, b):
            bb = base_b + step // t_steps
            t0 = (step % t_steps) * _R
            return pltpu.make_async_copy(
                out_v.at[b], out_hbm.at[bb, :, pl.ds(t0, _R)], sout[b])

        for b in range(_NBUF):
            in_copy(b, b).start()

        @pl.loop(0, steps, step=_NBUF)
        def _block(g):
            for b in range(_NBUF):
                step = g + b
                in_copy(step, b).wait()

                @pl.when(step >= _NBUF)
                def _():
                    out_copy(step - _NBUF, b).wait()

                @pl.loop(0, _R, unroll=4)
                def _row(r):
                    for s in range(_C // _LANES):
                        off = s * _LANES
                        for gi, grp in enumerate(_GROUPS):
                            acc = in_v[b, grp[0], r, pl.ds(off, _LANES)]
                            for j in grp[1:]:
                                acc = acc + in_v[b, j, r, pl.ds(off, _LANES)]
                            out_v[b, gi, r, pl.ds(off, _LANES)] = (
                                acc * (1.0 / len(grp)))

                out_copy(step, b).start()

                @pl.when(step + _NBUF < steps)
                def _():
                    in_copy(step + _NBUF, b).start()

        for b in range(_NBUF):
            out_copy(steps - _NBUF + b, b).wait()

    return k


def kernel(data2):
    B, T, J, C = data2.shape
    x = jnp.transpose(data2, (0, 2, 1, 3))
    out = _make_kernel(B, T)(x)
    return jnp.transpose(out, (0, 2, 1, 3))


# speedup vs baseline: 2.3345x; 2.3345x over previous
import functools

import jax
import jax.numpy as jnp
from jax import lax
from jax.experimental import pallas as pl
from jax.experimental.pallas import tpu as pltpu
from jax.experimental.pallas import tpu_sc as plsc

_GROUPS = ((1, 2), (3, 4), (5, 6), (7, 8), (0, 9),
           (10, 11, 12), (13, 14), (15, 16), (17, 18), (19, 20))
_J = 21
_G = 10
_C = 128
_LANES = 16
_NW = 32
_R = 8
_NBUF = 4


def _make_kernel(B, T):
    b_per_w = B // _NW
    t_steps = T // _R
    steps = b_per_w * t_steps
    assert steps % _NBUF == 0
    mesh = plsc.VectorSubcoreMesh(core_axis_name="c", subcore_axis_name="s")

    @functools.partial(
        pl.kernel,
        out_type=jax.ShapeDtypeStruct((B, _G, T, _C), jnp.float32),
        mesh=mesh,
        scratch_types=[
            pltpu.VMEM((_NBUF, _J, _R, _C), jnp.float32),
            pltpu.VMEM((_NBUF, _G, _R, _C), jnp.float32),
        ] + [pltpu.SemaphoreType.DMA] * (2 * _NBUF),
    )
    def k(x_hbm, out_hbm, in_v, out_v, *sems):
        sin = sems[:_NBUF]
        sout = sems[_NBUF:]
        wid = lax.axis_index("s") * 2 + lax.axis_index("c")
        base_b = wid * b_per_w

        def in_copy(step, b):
            bb = base_b + step // t_steps
            t0 = (step % t_steps) * _R
            return pltpu.make_async_copy(
                x_hbm.at[bb, :, pl.ds(t0, _R)], in_v.at[b], sin[b])

        def out_copy(step, b):
            bb = base_b + step // t_steps
            t0 = (step % t_steps) * _R
            return pltpu.make_async_copy(
                out_v.at[b], out_hbm.at[bb, :, pl.ds(t0, _R)], sout[b])

        for b in range(_NBUF):
            in_copy(b, b).start()

        @pl.loop(0, steps, step=_NBUF)
        def _block(g):
            for b in range(_NBUF):
                step = g + b
                in_copy(step, b).wait()

                @pl.when(step >= _NBUF)
                def _():
                    out_copy(step - _NBUF, b).wait()

                @pl.loop(0, _R)
                def _row(r):
                    for s in range(_C // _LANES):
                        off = s * _LANES
                        for gi, grp in enumerate(_GROUPS):
                            acc = in_v[b, grp[0], r, pl.ds(off, _LANES)]
                            for j in grp[1:]:
                                acc = acc + in_v[b, j, r, pl.ds(off, _LANES)]
                            out_v[b, gi, r, pl.ds(off, _LANES)] = (
                                acc * (1.0 / len(grp)))

                out_copy(step, b).start()

                @pl.when(step + _NBUF < steps)
                def _():
                    in_copy(step + _NBUF, b).start()

        for b in range(_NBUF):
            out_copy(steps - _NBUF + b, b).wait()

    return k


def kernel(data2):
    B, T, J, C = data2.shape
    x = jnp.transpose(data2, (0, 2, 1, 3))
    out = _make_kernel(B, T)(x)
    return jnp.transpose(out, (0, 2, 1, 3))


# speedup vs baseline: 2.3428x; 1.0035x over previous
import functools

import jax
import jax.numpy as jnp
from jax import lax
from jax.experimental import pallas as pl
from jax.experimental.pallas import tpu as pltpu
from jax.experimental.pallas import tpu_sc as plsc

_GROUPS = ((1, 2), (3, 4), (5, 6), (7, 8), (0, 9),
           (10, 11, 12), (13, 14), (15, 16), (17, 18), (19, 20))
_J = 21
_G = 10
_C = 128
_LANES = 16
_NW = 32
_R = 8
_NBUF = 4


def _make_kernel(B, T):
    b_per_w = B // _NW
    t_steps = T // _R
    steps = b_per_w * t_steps
    assert steps % _NBUF == 0
    mesh = plsc.VectorSubcoreMesh(core_axis_name="c", subcore_axis_name="s")

    @functools.partial(
        pl.kernel,
        out_type=jax.ShapeDtypeStruct((B, _G, T, _C), jnp.float32),
        mesh=mesh,
        scratch_types=[
            pltpu.VMEM((_NBUF, _J, _R, _C), jnp.float32),
            pltpu.VMEM((_NBUF, _G, _R, _C), jnp.float32),
        ] + [pltpu.SemaphoreType.DMA] * (3 * _NBUF),
    )
    def k(x_hbm, out_hbm, in_v, out_v, *sems):
        sin = sems[:_NBUF]
        sin2 = sems[_NBUF:2 * _NBUF]
        sout = sems[2 * _NBUF:]
        wid = lax.axis_index("s") * 2 + lax.axis_index("c")
        base_b = wid * b_per_w

        def in_copies(step, b):
            bb = base_b + step // t_steps
            t0 = (step % t_steps) * _R
            return (
                pltpu.make_async_copy(
                    x_hbm.at[bb, pl.ds(0, 11), pl.ds(t0, _R)],
                    in_v.at[b, pl.ds(0, 11)], sin[b]),
                pltpu.make_async_copy(
                    x_hbm.at[bb, pl.ds(11, 10), pl.ds(t0, _R)],
                    in_v.at[b, pl.ds(11, 10)], sin2[b]),
            )

        def out_copy(step, b):
            bb = base_b + step // t_steps
            t0 = (step % t_steps) * _R
            return pltpu.make_async_copy(
                out_v.at[b], out_hbm.at[bb, :, pl.ds(t0, _R)], sout[b])

        for b in range(_NBUF):
            for c in in_copies(b, b):
                c.start()

        @pl.loop(0, steps, step=_NBUF)
        def _block(g):
            for b in range(_NBUF):
                step = g + b
                for c in in_copies(step, b):
                    c.wait()

                @pl.when(step >= _NBUF)
                def _():
                    out_copy(step - _NBUF, b).wait()

                @pl.loop(0, _R)
                def _row(r):
                    for s in range(_C // _LANES):
                        off = s * _LANES
                        for gi, grp in enumerate(_GROUPS):
                            acc = in_v[b, grp[0], r, pl.ds(off, _LANES)]
                            for j in grp[1:]:
                                acc = acc + in_v[b, j, r, pl.ds(off, _LANES)]
                            out_v[b, gi, r, pl.ds(off, _LANES)] = (
                                acc * (1.0 / len(grp)))

                out_copy(step, b).start()

                @pl.when(step + _NBUF < steps)
                def _():
                    for c in in_copies(step + _NBUF, b):
                        c.start()

        for b in range(_NBUF):
            out_copy(steps - _NBUF + b, b).wait()

    return k


def kernel(data2):
    B, T, J, C = data2.shape
    x = jnp.transpose(data2, (0, 2, 1, 3))
    out = _make_kernel(B, T)(x)
    return jnp.transpose(out, (0, 2, 1, 3))
